# baseline (speedup 1.0000x reference)
import functools

import numpy as np
import jax
import jax.numpy as jnp
from jax import lax
from jax.experimental import pallas as pl
from jax.experimental.pallas import tpu as pltpu
from jax.experimental.pallas import tpu_sc as plsc

EPS = 1e-5


def _node_mm_body(x_ref, w_ref, b0_ref, ya_ref, yb_ref):
    d = x_ref.shape[1]
    xv = x_ref[...]
    ya_ref[...] = jnp.dot(xv, w_ref[:d, :], preferred_element_type=jnp.float32, precision=lax.Precision.HIGHEST)
    yb_ref[...] = (
        jnp.dot(xv, w_ref[d:, :], preferred_element_type=jnp.float32, precision=lax.Precision.HIGHEST)
        + b0_ref[...]
    )


def _node_mm(x, w0, b0):
    n, _ = x.shape
    h = w0.shape[1]
    out = jax.ShapeDtypeStruct((n, h), jnp.float32)
    return pl.pallas_call(
        _node_mm_body,
        out_shape=(out, out),
    )(x, w0, b0.reshape(1, h))


def _make_gather(rows_total, chunk, h, nc, ns):
    nw = nc * ns
    iters = (rows_total + nw - 1) // nw
    mesh = plsc.VectorSubcoreMesh(core_axis_name="c", subcore_axis_name="s")

    @functools.partial(
        pl.kernel,
        out_type=jax.ShapeDtypeStruct((rows_total, chunk, h), jnp.float32),
        mesh=mesh,
        scratch_types=[
            pltpu.VMEM((chunk,), jnp.int32),
            pltpu.VMEM((chunk,), jnp.int32),
            pltpu.VMEM((chunk, h), jnp.float32),
            pltpu.SemaphoreType.DMA,
        ],
        compiler_params=pltpu.CompilerParams(use_tc_tiling_on_sc=False),
    )
    def gather_kernel(ya_hbm, yb_hbm, s_hbm, e_hbm, g_hbm, idx_s, idx_e, rows, sem):
        wid = lax.axis_index("s") * nc + lax.axis_index("c")

        def body(j, carry):
            row = wid + nw * j

            @pl.when(row < rows_total)
            def _():
                pltpu.sync_copy(s_hbm.at[row], idx_s)
                pltpu.sync_copy(e_hbm.at[row], idx_e)
                pltpu.async_copy(ya_hbm.at[idx_s], rows, sem).wait()
                pltpu.async_copy(yb_hbm.at[idx_e], rows, sem, add=True).wait()
                pltpu.sync_copy(rows, g_hbm.at[row])

            return carry

        lax.fori_loop(0, iters, body, 0)

    return gather_kernel


def _exact_halfsum(a, mask):
    del mask

    def tree(v):
        w = v.shape[1]
        while w > 1:
            v = v[:, : w // 2] + v[:, w // 2 :]
            w //= 2
        return v

    h = a.shape[1] // 2
    return jnp.concatenate([tree(a[:, :h]), tree(a[:, h:])], axis=1)


def _bcast_halves(m, h):
    return jnp.concatenate(
        [jnp.broadcast_to(m[:, 0:1], (m.shape[0], h)),
         jnp.broadcast_to(m[:, 1:2], (m.shape[0], h))],
        axis=1,
    )


def _layer_norm_packed(hv, mask, lnw, lnb):
    h = mask.shape[0] // 2
    m = _exact_halfsum(hv, mask) * (1.0 / h)
    d = hv - _bcast_halves(m, h)
    v = _exact_halfsum(d * d, mask) * (1.0 / h)
    rs = 1.0 / jnp.sqrt(v + EPS)
    return d * _bcast_halves(rs, h) * lnw + lnb


def _stats_body(g_ref, mask_ref, lnw_ref, lnb_ref, o_ref):
    i = pl.program_id(0)
    hln = _layer_norm_packed(g_ref[...], mask_ref[...], lnw_ref[...], lnb_ref[...])
    blk = jnp.stack([jnp.sum(hln, axis=0), jnp.sum(hln * hln, axis=0)])

    @pl.when(i == 0)
    def _():
        o_ref[...] = blk

    @pl.when(i > 0)
    def _():
        o_ref[...] += blk


def _stats(g2, mask, lnw2, lnb2, blk_rows):
    r, h2 = g2.shape
    nb = r // blk_rows
    full = lambda i: (0, 0)
    return pl.pallas_call(
        _stats_body,
        grid=(nb,),
        in_specs=[
            pl.BlockSpec((blk_rows, h2), lambda i: (i, 0)),
            pl.BlockSpec((h2, 2), full),
            pl.BlockSpec((1, h2), full),
            pl.BlockSpec((1, h2), full),
        ],
        out_specs=pl.BlockSpec((2, h2), full),
        out_shape=jax.ShapeDtypeStruct((2, h2), jnp.float32),
    )(g2, mask, lnw2, lnb2)


def _make_apply_body(n_edges):
    inv_e = 1.0 / float(n_edges)

    def _apply_body(g_ref, stats_ref, mask_ref, lnw_ref, lnb_ref,
                    bnw_ref, bnb_ref, w1_ref, b1_ref, o_ref):
        hln = _layer_norm_packed(g_ref[...], mask_ref[...], lnw_ref[...],
                                 lnb_ref[...])

        st = stats_ref[...]
        h = mask_ref.shape[0] // 2
        tot = st + jnp.concatenate([st[:, h:], st[:, :h]], axis=1)
        bmean = tot[0:1, :] * inv_e
        bvar = tot[1:2, :] * inv_e - bmean * bmean
        hbn = (hln - bmean) / jnp.sqrt(bvar + EPS) * bnw_ref[...] + bnb_ref[...]
        s = hbn * jax.nn.sigmoid(hbn)
        o_ref[...] = _exact_halfsum(s * w1_ref[...], None) + b1_ref[...]

    return _apply_body


def _apply(g2, stats, mask, lnw2, lnb2, bnw2, bnb2, w1d, b1, blk_rows):
    r, h2 = g2.shape
    nb = r // blk_rows
    full = lambda i: (0, 0)
    return pl.pallas_call(
        _make_apply_body(2 * r),
        grid=(nb,),
        in_specs=[
            pl.BlockSpec((blk_rows, h2), lambda i: (i, 0)),
            pl.BlockSpec((2, h2), full),
            pl.BlockSpec((h2, 2), full),
            pl.BlockSpec((1, h2), full),
            pl.BlockSpec((1, h2), full),
            pl.BlockSpec((1, h2), full),
            pl.BlockSpec((1, h2), full),
            pl.BlockSpec((1, h2), full),
            pl.BlockSpec((1, 1), full),
        ],
        out_specs=pl.BlockSpec((blk_rows, 2), lambda i: (i, 0)),
        out_shape=jax.ShapeDtypeStruct((r, 2), jnp.float32),
    )(g2, stats, mask, lnw2, lnb2, bnw2, bnb2, w1d, b1.reshape(1, 1))


def kernel(x, edge_index, W0, b0, ln0_w, ln0_b, bn0_w, bn0_b, W1, b1):
    n, d = x.shape
    e = edge_index.shape[1]
    h = W0.shape[1]
    h2 = 2 * h
    chunk = 128
    rows_total = e // chunk

    start = edge_index[0].astype(jnp.int32).reshape(rows_total, chunk)
    end = edge_index[1].astype(jnp.int32).reshape(rows_total, chunk)

    ya, yb = _node_mm(x, W0, b0)

    info = plsc.get_sparse_core_info()
    g3 = _make_gather(rows_total, chunk, h, info.num_cores, info.num_subcores)(
        ya, yb, start, end
    )
    g2 = g3.reshape(e // 2, h2)

    mask = jnp.asarray(np.kron(np.eye(2, dtype=np.float32), np.ones((h, 1), np.float32)))
    dup = lambda p: jnp.concatenate([p, p]).reshape(1, h2)

    blk_rows = 8000
    stats = _stats(g2, mask, dup(ln0_w), dup(ln0_b), blk_rows)
    out2 = _apply(g2, stats, mask, dup(ln0_w), dup(ln0_b), dup(bn0_w),
                  dup(bn0_b), dup(W1[:, 0]), b1, blk_rows)
    return out2.reshape(e)


# speedup vs baseline: 2.1573x; 2.1573x over previous
import functools

import numpy as np
import jax
import jax.numpy as jnp
from jax import lax
from jax.experimental import pallas as pl
from jax.experimental.pallas import tpu as pltpu
from jax.experimental.pallas import tpu_sc as plsc

EPS = 1e-5


def _node_mm_body(x_ref, w_ref, b0_ref, ya_ref, yb_ref):
    d = x_ref.shape[1]
    xv = x_ref[...]
    ya_ref[...] = jnp.dot(xv, w_ref[:d, :], preferred_element_type=jnp.float32, precision=lax.Precision.HIGHEST)
    yb_ref[...] = (
        jnp.dot(xv, w_ref[d:, :], preferred_element_type=jnp.float32, precision=lax.Precision.HIGHEST)
        + b0_ref[...]
    )


def _node_mm(x, w0, b0):
    n, _ = x.shape
    h = w0.shape[1]
    out = jax.ShapeDtypeStruct((n, h), jnp.float32)
    return pl.pallas_call(
        _node_mm_body,
        out_shape=(out, out),
    )(x, w0, b0.reshape(1, h))


def _make_gather(rows_total, chunk, h, nc, ns):
    nw = nc * ns
    iters = (rows_total + nw - 1) // nw
    mesh = plsc.VectorSubcoreMesh(core_axis_name="c", subcore_axis_name="s")

    @functools.partial(
        pl.kernel,
        out_type=jax.ShapeDtypeStruct((rows_total, chunk, h), jnp.float32),
        mesh=mesh,
        scratch_types=[
            pltpu.VMEM((chunk,), jnp.int32),
            pltpu.VMEM((chunk,), jnp.int32),
            pltpu.VMEM((chunk, h), jnp.float32),
            pltpu.SemaphoreType.DMA,
        ],
        compiler_params=pltpu.CompilerParams(use_tc_tiling_on_sc=False),
    )
    def gather_kernel(ya_hbm, yb_hbm, s_hbm, e_hbm, g_hbm, idx_s, idx_e, rows, sem):
        wid = lax.axis_index("s") * nc + lax.axis_index("c")

        def body(j, carry):
            row = wid + nw * j

            @pl.when(row < rows_total)
            def _():
                pltpu.sync_copy(s_hbm.at[row], idx_s)
                pltpu.sync_copy(e_hbm.at[row], idx_e)
                pltpu.async_copy(ya_hbm.at[idx_s], rows, sem).wait()
                pltpu.async_copy(yb_hbm.at[idx_e], rows, sem, add=True).wait()
                pltpu.sync_copy(rows, g_hbm.at[row])

            return carry

        lax.fori_loop(0, iters, body, 0)

    return gather_kernel


def _halfsum(a, mask):
    return jnp.dot(a, mask, preferred_element_type=jnp.float32)


def _bcast_halves(m, h):
    return jnp.concatenate(
        [jnp.broadcast_to(m[:, 0:1], (m.shape[0], h)),
         jnp.broadcast_to(m[:, 1:2], (m.shape[0], h))],
        axis=1,
    )


def _layer_norm_packed(hv, mask, lnw, lnb):
    h = mask.shape[0] // 2
    m = _halfsum(hv, mask) * (1.0 / h)
    d = hv - _bcast_halves(m, h)
    v = _halfsum(d * d, mask) * (1.0 / h)
    rs = 1.0 / jnp.sqrt(v + EPS)
    return d * _bcast_halves(rs, h) * lnw + lnb


def _stats_body(g_ref, mask_ref, lnw_ref, lnb_ref, o_ref):
    i = pl.program_id(0)
    hln = _layer_norm_packed(g_ref[...], mask_ref[...], lnw_ref[...], lnb_ref[...])
    blk = jnp.stack([jnp.sum(hln, axis=0), jnp.sum(hln * hln, axis=0)])

    @pl.when(i == 0)
    def _():
        o_ref[...] = blk

    @pl.when(i > 0)
    def _():
        o_ref[...] += blk


def _stats(g2, mask, lnw2, lnb2, blk_rows):
    r, h2 = g2.shape
    nb = r // blk_rows
    full = lambda i: (0, 0)
    return pl.pallas_call(
        _stats_body,
        grid=(nb,),
        in_specs=[
            pl.BlockSpec((blk_rows, h2), lambda i: (i, 0)),
            pl.BlockSpec((h2, 2), full),
            pl.BlockSpec((1, h2), full),
            pl.BlockSpec((1, h2), full),
        ],
        out_specs=pl.BlockSpec((2, h2), full),
        out_shape=jax.ShapeDtypeStruct((2, h2), jnp.float32),
    )(g2, mask, lnw2, lnb2)


def _make_apply_body(n_edges):
    inv_e = 1.0 / float(n_edges)

    def _apply_body(g_ref, stats_ref, mask_ref, lnw_ref, lnb_ref,
                    bnw_ref, bnb_ref, w1_ref, b1_ref, o_ref):
        hln = _layer_norm_packed(g_ref[...], mask_ref[...], lnw_ref[...],
                                 lnb_ref[...])

        st = stats_ref[...]
        h = mask_ref.shape[0] // 2
        tot = st + jnp.concatenate([st[:, h:], st[:, :h]], axis=1)
        bmean = tot[0:1, :] * inv_e
        bvar = tot[1:2, :] * inv_e - bmean * bmean
        hbn = (hln - bmean) / jnp.sqrt(bvar + EPS) * bnw_ref[...] + bnb_ref[...]
        s = hbn * jax.nn.sigmoid(hbn)
        o_ref[...] = _halfsum(s * w1_ref[...], mask_ref[...]) + b1_ref[...]

    return _apply_body


def _apply(g2, stats, mask, lnw2, lnb2, bnw2, bnb2, w1d, b1, blk_rows):
    r, h2 = g2.shape
    nb = r // blk_rows
    full = lambda i: (0, 0)
    return pl.pallas_call(
        _make_apply_body(2 * r),
        grid=(nb,),
        in_specs=[
            pl.BlockSpec((blk_rows, h2), lambda i: (i, 0)),
            pl.BlockSpec((2, h2), full),
            pl.BlockSpec((h2, 2), full),
            pl.BlockSpec((1, h2), full),
            pl.BlockSpec((1, h2), full),
            pl.BlockSpec((1, h2), full),
            pl.BlockSpec((1, h2), full),
            pl.BlockSpec((1, h2), full),
            pl.BlockSpec((1, 1), full),
        ],
        out_specs=pl.BlockSpec((blk_rows, 2), lambda i: (i, 0)),
        out_shape=jax.ShapeDtypeStruct((r, 2), jnp.float32),
    )(g2, stats, mask, lnw2, lnb2, bnw2, bnb2, w1d, b1.reshape(1, 1))


def kernel(x, edge_index, W0, b0, ln0_w, ln0_b, bn0_w, bn0_b, W1, b1):
    n, d = x.shape
    e = edge_index.shape[1]
    h = W0.shape[1]
    h2 = 2 * h
    chunk = 128
    rows_total = e // chunk

    start = edge_index[0].astype(jnp.int32).reshape(rows_total, chunk)
    end = edge_index[1].astype(jnp.int32).reshape(rows_total, chunk)

    ya, yb = _node_mm(x, W0, b0)

    info = plsc.get_sparse_core_info()
    g3 = _make_gather(rows_total, chunk, h, info.num_cores, info.num_subcores)(
        ya, yb, start, end
    )
    g2 = g3.reshape(e // 2, h2)

    mask = jnp.asarray(np.kron(np.eye(2, dtype=np.float32), np.ones((h, 1), np.float32)))
    dup = lambda p: jnp.concatenate([p, p]).reshape(1, h2)

    blk_rows = 8000
    stats = _stats(g2, mask, dup(ln0_w), dup(ln0_b), blk_rows)
    out2 = _apply(g2, stats, mask, dup(ln0_w), dup(ln0_b), dup(bn0_w),
                  dup(bn0_b), dup(W1[:, 0]), b1, blk_rows)
    return out2.reshape(e)


# speedup vs baseline: 2.3747x; 1.1008x over previous
import functools

import numpy as np
import jax
import jax.numpy as jnp
from jax import lax
from jax.experimental import pallas as pl
from jax.experimental.pallas import tpu as pltpu
from jax.experimental.pallas import tpu_sc as plsc

EPS = 1e-5


def _node_mm_body(x_ref, w_ref, b0_ref, ya_ref, yb_ref):
    d = x_ref.shape[1]
    xv = x_ref[...]
    ya_ref[...] = jnp.dot(xv, w_ref[:d, :], preferred_element_type=jnp.float32, precision=lax.Precision.HIGHEST)
    yb_ref[...] = (
        jnp.dot(xv, w_ref[d:, :], preferred_element_type=jnp.float32, precision=lax.Precision.HIGHEST)
        + b0_ref[...]
    )


def _node_mm(x, w0, b0):
    n, _ = x.shape
    h = w0.shape[1]
    out = jax.ShapeDtypeStruct((n, h), jnp.float32)
    return pl.pallas_call(
        _node_mm_body,
        out_shape=(out, out),
    )(x, w0, b0.reshape(1, h))


def _make_gather(rows_total, chunk, h, nc, ns):
    nw = nc * ns
    iters = (rows_total + nw - 1) // nw
    mesh = plsc.VectorSubcoreMesh(core_axis_name="c", subcore_axis_name="s")

    pairs = (iters + 1) // 2

    @functools.partial(
        pl.kernel,
        out_type=jax.ShapeDtypeStruct((rows_total, chunk, h), jnp.float32),
        mesh=mesh,
        scratch_types=[
            pltpu.VMEM((chunk,), jnp.int32),
            pltpu.VMEM((chunk,), jnp.int32),
            pltpu.VMEM((chunk,), jnp.int32),
            pltpu.VMEM((chunk,), jnp.int32),
            pltpu.VMEM((chunk, h), jnp.float32),
            pltpu.VMEM((chunk, h), jnp.float32),
            pltpu.SemaphoreType.DMA,
            pltpu.SemaphoreType.DMA,
            pltpu.SemaphoreType.DMA,
            pltpu.SemaphoreType.DMA,
        ],
        compiler_params=pltpu.CompilerParams(use_tc_tiling_on_sc=False),
    )
    def gather_kernel(ya_hbm, yb_hbm, s_hbm, e_hbm, g_hbm,
                      idxs0, idxs1, idxe0, idxe1, rows0, rows1,
                      sem_i, sem_g, sem_w0, sem_w1):
        wid = lax.axis_index("s") * nc + lax.axis_index("c")
        bufs = ((idxs0, idxe0, rows0, sem_w0), (idxs1, idxe1, rows1, sem_w1))

        def pair_body(jj, carry):
            for k in range(2):
                idx_s, idx_e, rows, sem_w = bufs[k]
                row = wid + nw * (2 * jj + k)

                @pl.when(row < rows_total)
                def _():
                    ca = pltpu.async_copy(s_hbm.at[row], idx_s, sem_i)
                    cb = pltpu.async_copy(e_hbm.at[row], idx_e, sem_i)

                    @pl.when(jj > 0)
                    def _():
                        pltpu.make_async_copy(g_hbm.at[0], rows, sem_w).wait()

                    ca.wait()
                    cb.wait()
                    pltpu.async_copy(ya_hbm.at[idx_s], rows, sem_g).wait()
                    pltpu.async_copy(yb_hbm.at[idx_e], rows, sem_g, add=True).wait()
                    pltpu.async_copy(rows, g_hbm.at[row], sem_w)

            return carry

        lax.fori_loop(0, pairs, pair_body, 0)
        pltpu.make_async_copy(g_hbm.at[0], rows0, sem_w0).wait()
        pltpu.make_async_copy(g_hbm.at[0], rows1, sem_w1).wait()

    return gather_kernel


def _halfsum(a, mask):
    return jnp.dot(a, mask, preferred_element_type=jnp.float32)


def _bcast_halves(m, h):
    return jnp.concatenate(
        [jnp.broadcast_to(m[:, 0:1], (m.shape[0], h)),
         jnp.broadcast_to(m[:, 1:2], (m.shape[0], h))],
        axis=1,
    )


def _layer_norm_packed(hv, mask, lnw, lnb):
    h = mask.shape[0] // 2
    m = _halfsum(hv, mask) * (1.0 / h)
    d = hv - _bcast_halves(m, h)
    v = _halfsum(d * d, mask) * (1.0 / h)
    rs = 1.0 / jnp.sqrt(v + EPS)
    return d * _bcast_halves(rs, h) * lnw + lnb


def _stats_body(g_ref, mask_ref, lnw_ref, lnb_ref, o_ref):
    i = pl.program_id(0)
    hln = _layer_norm_packed(g_ref[...], mask_ref[...], lnw_ref[...], lnb_ref[...])
    blk = jnp.stack([jnp.sum(hln, axis=0), jnp.sum(hln * hln, axis=0)])

    @pl.when(i == 0)
    def _():
        o_ref[...] = blk

    @pl.when(i > 0)
    def _():
        o_ref[...] += blk


def _stats(g2, mask, lnw2, lnb2, blk_rows):
    r, h2 = g2.shape
    nb = r // blk_rows
    full = lambda i: (0, 0)
    return pl.pallas_call(
        _stats_body,
        grid=(nb,),
        in_specs=[
            pl.BlockSpec((blk_rows, h2), lambda i: (i, 0)),
            pl.BlockSpec((h2, 2), full),
            pl.BlockSpec((1, h2), full),
            pl.BlockSpec((1, h2), full),
        ],
        out_specs=pl.BlockSpec((2, h2), full),
        out_shape=jax.ShapeDtypeStruct((2, h2), jnp.float32),
    )(g2, mask, lnw2, lnb2)


def _make_apply_body(n_edges):
    inv_e = 1.0 / float(n_edges)

    def _apply_body(g_ref, stats_ref, mask_ref, lnw_ref, lnb_ref,
                    bnw_ref, bnb_ref, w1_ref, b1_ref, o_ref):
        hln = _layer_norm_packed(g_ref[...], mask_ref[...], lnw_ref[...],
                                 lnb_ref[...])

        st = stats_ref[...]
        h = mask_ref.shape[0] // 2
        tot = st + jnp.concatenate([st[:, h:], st[:, :h]], axis=1)
        bmean = tot[0:1, :] * inv_e
        bvar = tot[1:2, :] * inv_e - bmean * bmean
        hbn = (hln - bmean) / jnp.sqrt(bvar + EPS) * bnw_ref[...] + bnb_ref[...]
        s = hbn * jax.nn.sigmoid(hbn)
        o_ref[...] = _halfsum(s * w1_ref[...], mask_ref[...]) + b1_ref[...]

    return _apply_body


def _apply(g2, stats, mask, lnw2, lnb2, bnw2, bnb2, w1d, b1, blk_rows):
    r, h2 = g2.shape
    nb = r // blk_rows
    full = lambda i: (0, 0)
    return pl.pallas_call(
        _make_apply_body(2 * r),
        grid=(nb,),
        in_specs=[
            pl.BlockSpec((blk_rows, h2), lambda i: (i, 0)),
            pl.BlockSpec((2, h2), full),
            pl.BlockSpec((h2, 2), full),
            pl.BlockSpec((1, h2), full),
            pl.BlockSpec((1, h2), full),
            pl.BlockSpec((1, h2), full),
            pl.BlockSpec((1, h2), full),
            pl.BlockSpec((1, h2), full),
            pl.BlockSpec((1, 1), full),
        ],
        out_specs=pl.BlockSpec((blk_rows, 2), lambda i: (i, 0)),
        out_shape=jax.ShapeDtypeStruct((r, 2), jnp.float32),
    )(g2, stats, mask, lnw2, lnb2, bnw2, bnb2, w1d, b1.reshape(1, 1))


def kernel(x, edge_index, W0, b0, ln0_w, ln0_b, bn0_w, bn0_b, W1, b1):
    n, d = x.shape
    e = edge_index.shape[1]
    h = W0.shape[1]
    h2 = 2 * h
    chunk = 128
    rows_total = e // chunk

    start = edge_index[0].astype(jnp.int32).reshape(rows_total, chunk)
    end = edge_index[1].astype(jnp.int32).reshape(rows_total, chunk)

    ya, yb = _node_mm(x, W0, b0)

    info = plsc.get_sparse_core_info()
    g3 = _make_gather(rows_total, chunk, h, info.num_cores, info.num_subcores)(
        ya, yb, start, end
    )
    g2 = g3.reshape(e // 2, h2)

    mask = jnp.asarray(np.kron(np.eye(2, dtype=np.float32), np.ones((h, 1), np.float32)))
    dup = lambda p: jnp.concatenate([p, p]).reshape(1, h2)

    blk_rows = 8000
    stats = _stats(g2, mask, dup(ln0_w), dup(ln0_b), blk_rows)
    out2 = _apply(g2, stats, mask, dup(ln0_w), dup(ln0_b), dup(bn0_w),
                  dup(bn0_b), dup(W1[:, 0]), b1, blk_rows)
    return out2.reshape(e)
